# baseline (speedup 1.0000x reference)
import functools

import jax
import jax.numpy as jnp
from jax import lax
from jax.experimental import pallas as pl
from jax.experimental.pallas import tpu as pltpu
from jax.experimental.pallas import tpu_sc as plsc

E = 8
K = 2
H = 1024
I = 2048
T = 4096

BM = 256
NB = (T * K) // BM + E
P = NB * BM

NC = 2
NS = 16
NW = NC * NS

NSPLIT = 2
P2 = P // NSPLIT
NB2 = NB // NSPLIT
G_CH = 32
G_NCH = P2 // (NW * G_CH)
G_ROWS_W = P2 // NW
G_NBUF = 3


def _sc_gather_body(x_hbm, ids_hbm, out_hbm, idx_v, buf0, buf1, buf2,
                    g0, g1, g2, w0, w1s, w2s):
    wid = lax.axis_index("s") * NC + lax.axis_index("c")
    rbase = wid * G_ROWS_W
    pltpu.sync_copy(ids_hbm.at[wid], idx_v)
    bufs = (buf0, buf1, buf2)
    gsems = (g0, g1, g2)
    wsems = (w0, w1s, w2s)
    gcp = [None] * G_NBUF
    wcp = [None] * G_NBUF
    for c in range(min(2, G_NCH)):
        gcp[c] = pltpu.async_copy(x_hbm.at[idx_v.at[c]], bufs[c], gsems[c])
    for c in range(G_NCH):
        b = c % G_NBUF
        gcp[b].wait()
        wcp[b] = pltpu.async_copy(
            bufs[b], out_hbm.at[pl.ds(rbase + c * G_CH, G_CH)], wsems[b]
        )
        if c + 2 < G_NCH:
            nb = (c + 2) % G_NBUF
            if wcp[nb] is not None:
                wcp[nb].wait()
            gcp[nb] = pltpu.async_copy(
                x_hbm.at[idx_v.at[c + 2]], bufs[nb], gsems[nb]
            )
    for b in range(G_NBUF):
        if wcp[b] is not None:
            wcp[b].wait()


C_CH = 16
C_NCH = T // (NW * C_CH)
C_TOK_W = T // NW


C_NBUF = 2


def _sc_combine_body(yw_hbm, i0_hbm, i1_hbm, w0_hbm, w1_hbm, out_hbm,
                     i0v, i1v, w0v, w1v, *rest):
    wid = lax.axis_index("s") * NC + lax.axis_index("c")
    tbase = wid * C_TOK_W
    pltpu.sync_copy(i0_hbm.at[wid], i0v)
    pltpu.sync_copy(i1_hbm.at[wid], i1v)
    pltpu.sync_copy(w0_hbm.at[wid], w0v)
    pltpu.sync_copy(w1_hbm.at[wid], w1v)
    abufs = rest[:C_NBUF]
    bbufs = rest[C_NBUF:2 * C_NBUF]
    asems = rest[2 * C_NBUF:3 * C_NBUF]
    bsems = rest[3 * C_NBUF:4 * C_NBUF]
    wsems = rest[4 * C_NBUF:5 * C_NBUF]
    acp = [None] * C_NBUF
    bcp = [None] * C_NBUF
    wcp = [None] * C_NBUF
    for c in range(min(2, C_NCH)):
        acp[c] = pltpu.async_copy(yw_hbm.at[i0v.at[c]], abufs[c], asems[c])
        bcp[c] = pltpu.async_copy(yw_hbm.at[i1v.at[c]], bbufs[c], bsems[c])
    for c in range(C_NCH):
        b = c % C_NBUF
        acp[b].wait()
        bcp[b].wait()
        a_buf, b_buf = abufs[b], bbufs[b]
        for r in range(C_CH):
            w0vec = w0v[c, r, :]
            w1vec = w1v[c, r, :]
            def _add(j, _, r=r, w0vec=w0vec, w1vec=w1vec):
                sl = pl.ds(j * 16, 16)
                a_buf[r, sl] = a_buf[r, sl] * w0vec + b_buf[r, sl] * w1vec
                return 0
            lax.fori_loop(0, H // 16, _add, 0)
        wcp[b] = pltpu.async_copy(
            a_buf, out_hbm.at[pl.ds(tbase + c * C_CH, C_CH)], wsems[b]
        )
        if c + 2 < C_NCH:
            nb = (c + 2) % C_NBUF
            if wcp[nb] is not None:
                wcp[nb].wait()
            acp[nb] = pltpu.async_copy(yw_hbm.at[i0v.at[c + 2]], abufs[nb], asems[nb])
            bcp[nb] = pltpu.async_copy(yw_hbm.at[i1v.at[c + 2]], bbufs[nb], bsems[nb])
    for b in range(C_NBUF):
        if wcp[b] is not None:
            wcp[b].wait()


@functools.lru_cache(maxsize=None)
def _sc_kernels():
    mesh = plsc.VectorSubcoreMesh(
        core_axis_name="c", subcore_axis_name="s", num_cores=NC, num_subcores=NS
    )
    gather = pl.kernel(
        _sc_gather_body,
        out_type=jax.ShapeDtypeStruct((P2, H), jnp.float32),
        mesh=mesh,
        scratch_types=[pltpu.VMEM((G_NCH, G_CH), jnp.int32)]
        + [pltpu.VMEM((G_CH, H), jnp.float32) for _ in range(G_NBUF)]
        + [pltpu.SemaphoreType.DMA] * (2 * G_NBUF),
    )
    combine = pl.kernel(
        _sc_combine_body,
        out_type=jax.ShapeDtypeStruct((T, H), jnp.float32),
        mesh=mesh,
        scratch_types=[pltpu.VMEM((C_NCH, C_CH), jnp.int32)] * 2
        + [pltpu.VMEM((C_NCH, C_CH, 16), jnp.float32)] * 2
        + [pltpu.VMEM((C_CH, H), jnp.float32) for _ in range(2 * C_NBUF)]
        + [pltpu.SemaphoreType.DMA] * (3 * C_NBUF),
    )
    return gather, combine


def _mlp_body(s_ref, x_ref, w1_ref, w3_ref, w2_ref, *rest):
    out_ref = rest[-1]
    x = x_ref[...].astype(jnp.bfloat16)
    w1 = w1_ref[0]
    w3 = w3_ref[0]
    w2 = w2_ref[0]
    dn = (((1,), (1,)), ((), ()))
    g = lax.dot_general(x, w1, dn, preferred_element_type=jnp.float32)
    u = lax.dot_general(x, w3, dn, preferred_element_type=jnp.float32)
    act = (g * jax.nn.sigmoid(g) * u).astype(jnp.bfloat16)
    y = lax.dot_general(act, w2, dn, preferred_element_type=jnp.float32)
    out_ref[...] = y


def _make_tc(off, alias):
    in_specs = [
        pl.BlockSpec((BM, H), lambda b, s: (b, 0)),
        pl.BlockSpec((1, I, H), lambda b, s: (s[b + off], 0, 0)),
        pl.BlockSpec((1, I, H), lambda b, s: (s[b + off], 0, 0)),
        pl.BlockSpec((1, H, I), lambda b, s: (s[b + off], 0, 0)),
    ]
    kwargs = {}
    if alias:
        in_specs.append(pl.BlockSpec(memory_space=pl.ANY))
        kwargs["input_output_aliases"] = {5: 0}
    return pl.pallas_call(
        _mlp_body,
        grid_spec=pltpu.PrefetchScalarGridSpec(
            num_scalar_prefetch=1,
            grid=(NB2,),
            in_specs=in_specs,
            out_specs=pl.BlockSpec((BM, H), lambda b, s: (b + off, 0)),
        ),
        out_shape=jax.ShapeDtypeStruct((P, H), jnp.float32),
        compiler_params=pltpu.CompilerParams(
            dimension_semantics=("arbitrary",),
            vmem_limit_bytes=100 * 1024 * 1024,
        ),
        **kwargs,
    )


_tc_mlp_a = _make_tc(0, alias=False)
_tc_mlp_b = _make_tc(NB2, alias=True)


def _plan(router_logits):
    probs = jax.nn.softmax(router_logits, axis=-1)
    rw, sel = lax.top_k(probs, K)
    flat_e = sel.reshape(-1).astype(jnp.int32)
    eids = jnp.arange(E, dtype=jnp.int32)
    oh = (flat_e[:, None] == eids[None, :]).astype(jnp.int32)
    counts = jnp.sum(oh, axis=0)
    pc = ((counts + BM - 1) // BM) * BM
    pad_end = jnp.cumsum(pc)
    pad_off = pad_end - pc
    off = jnp.cumsum(counts) - counts
    rank = jnp.sum(oh * jnp.cumsum(oh, axis=0), axis=1) - 1
    dst_pair = jnp.sum(oh * pad_off[None, :], axis=1) + rank
    inv = dst_pair.reshape(T, K)
    order = jnp.argsort(flat_e, stable=True)
    p = jnp.arange(P, dtype=jnp.int32)
    g_p = jnp.minimum(
        jnp.sum(p[:, None] >= pad_end[None, :], axis=1, dtype=jnp.int32), E - 1
    )
    goh = (g_p[:, None] == eids[None, :]).astype(jnp.int32)
    src_rank = p - jnp.sum(goh * pad_off[None, :], axis=1)
    valid = src_rank < jnp.sum(goh * counts[None, :], axis=1)
    src_sorted = jnp.clip(
        jnp.sum(goh * off[None, :], axis=1) + src_rank, 0, T * K - 1
    )
    pair = order[src_sorted].astype(jnp.int32)
    row_token = jnp.where(valid, pair // K, 0).astype(jnp.int32)
    block_start = jnp.arange(NB, dtype=jnp.int32) * BM
    block_expert = jnp.minimum(
        jnp.sum(block_start[:, None] >= pad_end[None, :], axis=1, dtype=jnp.int32),
        E - 1,
    )
    return row_token, rw.astype(jnp.float32), inv, block_expert


def kernel(hidden_states, router_logits, w1, w3, w2):
    x = hidden_states.reshape(T, H)
    row_token, rw, inv, block_expert = _plan(router_logits)

    w1b = w1.astype(jnp.bfloat16)
    w3b = w3.astype(jnp.bfloat16)
    w2b = w2.astype(jnp.bfloat16)
    sc_gather, sc_combine = _sc_kernels()
    rt = row_token.reshape(NSPLIT, NW, G_NCH, G_CH)
    x_pad0 = sc_gather(x, rt[0])
    x_pad1 = sc_gather(x, rt[1])
    yw0 = _tc_mlp_a(block_expert, x_pad0, w1b, w3b, w2b)
    yw = _tc_mlp_b(block_expert, x_pad1, w1b, w3b, w2b, yw0)
    wbc = jnp.broadcast_to(rw[:, :, None], (T, K, 16))
    out = sc_combine(
        yw,
        inv[:, 0].reshape(NW, C_NCH, C_CH),
        inv[:, 1].reshape(NW, C_NCH, C_CH),
        wbc[:, 0].reshape(NW, C_NCH, C_CH, 16),
        wbc[:, 1].reshape(NW, C_NCH, C_CH, 16),
    )
    return out


# speedup vs baseline: 1.2089x; 1.2089x over previous
import functools

import jax
import jax.numpy as jnp
from jax import lax
from jax.experimental import pallas as pl
from jax.experimental.pallas import tpu as pltpu
from jax.experimental.pallas import tpu_sc as plsc

E = 8
K = 2
H = 1024
I = 2048
T = 4096

BM = 256
NB = (T * K) // BM + E
P = NB * BM

NC = 2
NS = 16
NW = NC * NS

NSPLIT = 2
P2 = P // NSPLIT
NB2 = NB // NSPLIT
G_CH = 32
G_NCH = P2 // (NW * G_CH)
G_ROWS_W = P2 // NW
G_NBUF = 3


def _sc_gather_body(x_hbm, ids_hbm, out_hbm, idx_v, buf0, buf1, buf2,
                    g0, g1, g2, w0, w1s, w2s):
    wid = lax.axis_index("s") * NC + lax.axis_index("c")
    rbase = wid * G_ROWS_W
    pltpu.sync_copy(ids_hbm.at[wid], idx_v)
    bufs = (buf0, buf1, buf2)
    gsems = (g0, g1, g2)
    wsems = (w0, w1s, w2s)
    gcp = [None] * G_NBUF
    wcp = [None] * G_NBUF
    for c in range(min(2, G_NCH)):
        gcp[c] = pltpu.async_copy(x_hbm.at[idx_v.at[c]], bufs[c], gsems[c])
    for c in range(G_NCH):
        b = c % G_NBUF
        gcp[b].wait()
        wcp[b] = pltpu.async_copy(
            bufs[b], out_hbm.at[pl.ds(rbase + c * G_CH, G_CH)], wsems[b]
        )
        if c + 2 < G_NCH:
            nb = (c + 2) % G_NBUF
            if wcp[nb] is not None:
                wcp[nb].wait()
            gcp[nb] = pltpu.async_copy(
                x_hbm.at[idx_v.at[c + 2]], bufs[nb], gsems[nb]
            )
    for b in range(G_NBUF):
        if wcp[b] is not None:
            wcp[b].wait()


C_CH = 16
C_NCH = T // (NW * C_CH)
C_TOK_W = T // NW


C_NBUF = 2


def _sc_combine_body(yw_hbm, i0_hbm, i1_hbm, w0_hbm, w1_hbm, out_hbm,
                     i0v, i1v, w0v, w1v, *rest):
    wid = lax.axis_index("s") * NC + lax.axis_index("c")
    tbase = wid * C_TOK_W
    pltpu.sync_copy(i0_hbm.at[wid], i0v)
    pltpu.sync_copy(i1_hbm.at[wid], i1v)
    pltpu.sync_copy(w0_hbm.at[wid], w0v)
    pltpu.sync_copy(w1_hbm.at[wid], w1v)
    abufs = rest[:C_NBUF]
    bbufs = rest[C_NBUF:2 * C_NBUF]
    asems = rest[2 * C_NBUF:3 * C_NBUF]
    bsems = rest[3 * C_NBUF:4 * C_NBUF]
    wsems = rest[4 * C_NBUF:5 * C_NBUF]
    acp = [None] * C_NBUF
    bcp = [None] * C_NBUF
    wcp = [None] * C_NBUF
    for c in range(min(2, C_NCH)):
        acp[c] = pltpu.async_copy(yw_hbm.at[i0v.at[c]], abufs[c], asems[c])
        bcp[c] = pltpu.async_copy(yw_hbm.at[i1v.at[c]], bbufs[c], bsems[c])
    for c in range(C_NCH):
        b = c % C_NBUF
        acp[b].wait()
        bcp[b].wait()
        a_buf, b_buf = abufs[b], bbufs[b]
        for r in range(C_CH):
            w0vec = w0v[c, r, :]
            w1vec = w1v[c, r, :]
            def _add(j, _, r=r, w0vec=w0vec, w1vec=w1vec):
                sl = pl.ds(j * 16, 16)
                a_buf[r, sl] = a_buf[r, sl] * w0vec + b_buf[r, sl] * w1vec
                return 0
            lax.fori_loop(0, H // 16, _add, 0)
        wcp[b] = pltpu.async_copy(
            a_buf, out_hbm.at[pl.ds(tbase + c * C_CH, C_CH)], wsems[b]
        )
        if c + 2 < C_NCH:
            nb = (c + 2) % C_NBUF
            if wcp[nb] is not None:
                wcp[nb].wait()
            acp[nb] = pltpu.async_copy(yw_hbm.at[i0v.at[c + 2]], abufs[nb], asems[nb])
            bcp[nb] = pltpu.async_copy(yw_hbm.at[i1v.at[c + 2]], bbufs[nb], bsems[nb])
    for b in range(C_NBUF):
        if wcp[b] is not None:
            wcp[b].wait()


@functools.lru_cache(maxsize=None)
def _sc_kernels():
    mesh = plsc.VectorSubcoreMesh(
        core_axis_name="c", subcore_axis_name="s", num_cores=NC, num_subcores=NS
    )
    gather = pl.kernel(
        _sc_gather_body,
        out_type=jax.ShapeDtypeStruct((P2, H), jnp.float32),
        mesh=mesh,
        scratch_types=[pltpu.VMEM((G_NCH, G_CH), jnp.int32)]
        + [pltpu.VMEM((G_CH, H), jnp.float32) for _ in range(G_NBUF)]
        + [pltpu.SemaphoreType.DMA] * (2 * G_NBUF),
    )
    combine = pl.kernel(
        _sc_combine_body,
        out_type=jax.ShapeDtypeStruct((T, H), jnp.float32),
        mesh=mesh,
        scratch_types=[pltpu.VMEM((C_NCH, C_CH), jnp.int32)] * 2
        + [pltpu.VMEM((C_NCH, C_CH, 16), jnp.float32)] * 2
        + [pltpu.VMEM((C_CH, H), jnp.float32) for _ in range(2 * C_NBUF)]
        + [pltpu.SemaphoreType.DMA] * (3 * C_NBUF),
    )
    return gather, combine


def _mlp_body(s_ref, x_ref, w1_ref, w3_ref, w2_ref, *rest):
    out_ref = rest[-1]
    x = x_ref[...].astype(jnp.bfloat16)
    w1 = w1_ref[0].astype(jnp.bfloat16)
    w3 = w3_ref[0].astype(jnp.bfloat16)
    w2 = w2_ref[0].astype(jnp.bfloat16)
    dn = (((1,), (1,)), ((), ()))
    g = lax.dot_general(x, w1, dn, preferred_element_type=jnp.float32)
    u = lax.dot_general(x, w3, dn, preferred_element_type=jnp.float32)
    act = (g * jax.nn.sigmoid(g) * u).astype(jnp.bfloat16)
    y = lax.dot_general(act, w2, dn, preferred_element_type=jnp.float32)
    out_ref[...] = y


def _make_tc(off, alias):
    in_specs = [
        pl.BlockSpec((BM, H), lambda b, s: (b, 0)),
        pl.BlockSpec((1, I, H), lambda b, s: (s[b + off], 0, 0)),
        pl.BlockSpec((1, I, H), lambda b, s: (s[b + off], 0, 0)),
        pl.BlockSpec((1, H, I), lambda b, s: (s[b + off], 0, 0)),
    ]
    kwargs = {}
    if alias:
        in_specs.append(pl.BlockSpec(memory_space=pl.ANY))
        kwargs["input_output_aliases"] = {5: 0}
    return pl.pallas_call(
        _mlp_body,
        grid_spec=pltpu.PrefetchScalarGridSpec(
            num_scalar_prefetch=1,
            grid=(NB2,),
            in_specs=in_specs,
            out_specs=pl.BlockSpec((BM, H), lambda b, s: (b + off, 0)),
        ),
        out_shape=jax.ShapeDtypeStruct((P, H), jnp.float32),
        compiler_params=pltpu.CompilerParams(
            dimension_semantics=("arbitrary",),
            vmem_limit_bytes=100 * 1024 * 1024,
        ),
        **kwargs,
    )


_tc_mlp_a = _make_tc(0, alias=False)
_tc_mlp_b = _make_tc(NB2, alias=True)


def _plan(router_logits):
    probs = jax.nn.softmax(router_logits, axis=-1)
    rw, sel = lax.top_k(probs, K)
    flat_e = sel.reshape(-1).astype(jnp.int32)
    eids = jnp.arange(E, dtype=jnp.int32)
    oh = (flat_e[:, None] == eids[None, :]).astype(jnp.int32)
    counts = jnp.sum(oh, axis=0)
    pc = ((counts + BM - 1) // BM) * BM
    pad_end = jnp.cumsum(pc)
    pad_off = pad_end - pc
    off = jnp.cumsum(counts) - counts
    rank = jnp.sum(oh * jnp.cumsum(oh, axis=0), axis=1) - 1
    dst_pair = jnp.sum(oh * pad_off[None, :], axis=1) + rank
    inv = dst_pair.reshape(T, K)
    order = jnp.argsort(flat_e, stable=True)
    p = jnp.arange(P, dtype=jnp.int32)
    g_p = jnp.minimum(
        jnp.sum(p[:, None] >= pad_end[None, :], axis=1, dtype=jnp.int32), E - 1
    )
    goh = (g_p[:, None] == eids[None, :]).astype(jnp.int32)
    src_rank = p - jnp.sum(goh * pad_off[None, :], axis=1)
    valid = src_rank < jnp.sum(goh * counts[None, :], axis=1)
    src_sorted = jnp.clip(
        jnp.sum(goh * off[None, :], axis=1) + src_rank, 0, T * K - 1
    )
    pair = order[src_sorted].astype(jnp.int32)
    row_token = jnp.where(valid, pair // K, 0).astype(jnp.int32)
    block_start = jnp.arange(NB, dtype=jnp.int32) * BM
    block_expert = jnp.minimum(
        jnp.sum(block_start[:, None] >= pad_end[None, :], axis=1, dtype=jnp.int32),
        E - 1,
    )
    return row_token, rw.astype(jnp.float32), inv, block_expert


def kernel(hidden_states, router_logits, w1, w3, w2):
    x = hidden_states.reshape(T, H)
    row_token, rw, inv, block_expert = _plan(router_logits)

    sc_gather, sc_combine = _sc_kernels()
    rt = row_token.reshape(NSPLIT, NW, G_NCH, G_CH)
    x_pad0 = sc_gather(x, rt[0])
    x_pad1 = sc_gather(x, rt[1])
    yw0 = _tc_mlp_a(block_expert, x_pad0, w1, w3, w2)
    yw = _tc_mlp_b(block_expert, x_pad1, w1, w3, w2, yw0)
    wbc = jnp.broadcast_to(rw[:, :, None], (T, K, 16))
    out = sc_combine(
        yw,
        inv[:, 0].reshape(NW, C_NCH, C_CH),
        inv[:, 1].reshape(NW, C_NCH, C_CH),
        wbc[:, 0].reshape(NW, C_NCH, C_CH, 16),
        wbc[:, 1].reshape(NW, C_NCH, C_CH, 16),
    )
    return out
